# baseline (speedup 1.0000x reference)
import functools

import jax
import jax.numpy as jnp
from jax import lax
from jax.experimental import pallas as pl
from jax.experimental.pallas import tpu as pltpu
from jax.experimental.pallas import tpu_sc as plsc

_OTHER = 24
_BEH = 2
_VOCAB = 100000
_D = 16
_L = 16
_NS = 16


def _din_gather(seq_t, item_t, wseq_t, wbeh_t):
    mesh = plsc.VectorSubcoreMesh(core_axis_name="c", subcore_axis_name="s",
                                  num_cores=1)

    @functools.partial(
        pl.kernel,
        mesh=mesh,
        out_type=[
            jax.ShapeDtypeStruct((_OTHER * _D,), jnp.float32),
            jax.ShapeDtypeStruct((_BEH * _D,), jnp.float32),
        ],
        scratch_types=[
            pltpu.VMEM((_OTHER, 128), jnp.int32),
            pltpu.VMEM((_BEH, 128), jnp.int32),
            pltpu.VMEM((_D, 128), jnp.float32),
            pltpu.VMEM((_D, 128), jnp.float32),
            pltpu.VMEM((_D,), jnp.float32),
            pltpu.VMEM((_D,), jnp.float32),
            pltpu.SemaphoreType.DMA,
            pltpu.SemaphoreType.DMA,
            pltpu.SemaphoreType.DMA,
        ],
        compiler_params=pltpu.CompilerParams(
            needs_layout_passes=False, disable_bounds_checks=True),
    )
    def k(seq_hbm, item_hbm, wseq_hbm, wbeh_hbm, seq_out, beh_out,
          idbuf, bidbuf, blk0, blk1, row0, row1, sem0, sem1, osem):
        wid = lax.axis_index("s")
        lanes = lax.iota(jnp.int32, _L)
        zeros = lanes * 0

        def fire(ids_ref, f, table_hbm, blk, sem):
            idv = plsc.load_gather(ids_ref, [jnp.full((_L,), f, jnp.int32),
                                             zeros])
            base = lax.shift_left(lax.shift_right_logical(idv, 7), 7)
            col = idv & 127
            base_s = pl.multiple_of(jnp.max(base), 128)
            cp = pltpu.async_copy(
                table_hbm.at[f, pl.ds(0, _D), pl.ds(base_s, 128)], blk, sem)
            return cp, col

        def finish(cp, blk, col, row_v, f, out_ref):
            cp.wait()
            row_v[...] = plsc.load_gather(blk, [lanes, col])
            off = pl.multiple_of(f * _D, _D)
            return pltpu.async_copy(row_v, out_ref.at[pl.ds(off, _D)], osem)

        pltpu.sync_copy(seq_hbm.at[0, pl.ds(0, _OTHER), pl.ds(0, 128)],
                        idbuf)
        cp0, col0 = fire(idbuf, wid, wseq_hbm, blk0, sem0)

        @pl.when(wid < _OTHER + _BEH - _NS)
        def _second():
            j = wid + _NS

            @pl.when(j < _OTHER)
            def _seq2():
                cp1, col1 = fire(idbuf, j, wseq_hbm, blk1, sem1)
                finish(cp1, blk1, col1, row1, j, seq_out).wait()

            @pl.when(j >= _OTHER)
            def _beh2():
                pltpu.sync_copy(item_hbm.at[0, pl.ds(0, _BEH), pl.ds(0, 128)],
                                bidbuf)
                cp1, col1 = fire(bidbuf, j - _OTHER, wbeh_hbm, blk1, sem1)
                finish(cp1, blk1, col1, row1, j - _OTHER, beh_out).wait()

        finish(cp0, blk0, col0, row0, wid, seq_out).wait()

    return k(seq_t, item_t, wseq_t, wbeh_t)


@jax.jit
def kernel(dense_inputs, sparse_inputs, seq_inputs, item_inputs, W_seq, W_beh):
    del dense_inputs, sparse_inputs
    seq_t = seq_inputs.astype(jnp.int32).transpose(1, 2, 0)
    item_t = item_inputs.astype(jnp.int32).transpose(1, 2, 0)
    wseq_t = W_seq.transpose(0, 2, 1)
    wbeh_t = W_beh.transpose(0, 2, 1)
    seq_embed, behavior_embedded = _din_gather(seq_t, item_t, wseq_t, wbeh_t)
    return seq_embed, behavior_embedded
